# baseline (speedup 1.0000x reference)
import functools

import jax
import jax.numpy as jnp
import numpy as np
from jax import lax
from jax.experimental import pallas as pl
from jax.experimental.pallas import tpu as pltpu
from jax.experimental.pallas import tpu_sc as plsc

_POOLING_RATE = 4
_NEIGHBOR_NUM = 4
_ANCHOR = 12
_ROW = 16


def _knn_topk_tc(vertices, queries):
    bs, V, _ = vertices.shape
    Q = queries.shape[2]
    QB = 256

    def body(v_ref, q_ref, o_ref):
        wx = v_ref[0, :, 0:1]
        wy = v_ref[0, :, 1:2]
        wz = v_ref[0, :, 2:3]
        qx = q_ref[0, 0:1, :]
        qy = q_ref[0, 1:2, :]
        qz = q_ref[0, 2:3, :]
        wn = wx * wx + wy * wy + wz * wz
        qn = qx * qx + qy * qy + qz * qz
        wxb = wx.astype(jnp.bfloat16).astype(jnp.float32)
        wyb = wy.astype(jnp.bfloat16).astype(jnp.float32)
        wzb = wz.astype(jnp.bfloat16).astype(jnp.float32)
        qxb = qx.astype(jnp.bfloat16).astype(jnp.float32)
        qyb = qy.astype(jnp.bfloat16).astype(jnp.float32)
        qzb = qz.astype(jnp.bfloat16).astype(jnp.float32)
        inner = (wxb * qxb + wyb * qyb) + wzb * qzb
        dist = (inner * (-2.0) + wn) + qn
        iota = lax.broadcasted_iota(jnp.int32, (V, QB), 0)
        big = jnp.int32(2 ** 30)
        for k in range(5):
            mval = jnp.min(dist, axis=0, keepdims=True)
            cand = jnp.where(dist == mval, iota, big)
            midx = jnp.min(cand, axis=0, keepdims=True)
            if k > 0:
                o_ref[0, k:k + 1, :] = midx
            if k < 4:
                dist = jnp.where(iota == midx, jnp.float32(jnp.inf), dist)

    return pl.pallas_call(
        body,
        grid=(bs, Q // QB),
        in_specs=[
            pl.BlockSpec((1, V, 3), lambda b, i: (b, 0, 0)),
            pl.BlockSpec((1, 3, QB), lambda b, i: (b, 0, i)),
        ],
        out_specs=pl.BlockSpec((1, 8, QB), lambda b, i: (b, 0, i)),
        out_shape=jax.ShapeDtypeStruct((bs, 8, Q), jnp.int32),
    )(vertices, queries)


def _gather_max_sc(nbr_flat, fm_t, bs, C, V, P):
    info = plsc.get_sparse_core_info()
    NC, NS = info.num_cores, info.num_subcores
    NW = NC * NS
    CG = 8
    mesh = plsc.VectorSubcoreMesh(core_axis_name="c", subcore_axis_name="s")

    @functools.partial(
        pl.kernel,
        mesh=mesh,
        out_type=jax.ShapeDtypeStruct((bs, _ANCHOR, C, P), jnp.float32),
        compiler_params=pltpu.CompilerParams(use_tc_tiling_on_sc=True,
                                             needs_layout_passes=False),
        scratch_types=[
            pltpu.VMEM((_NEIGHBOR_NUM * P,), jnp.int32),
            pltpu.VMEM((CG, V), jnp.float32),
            pltpu.VMEM((CG, V), jnp.float32),
            pltpu.VMEM((CG, P), jnp.float32),
            pltpu.SemaphoreType.DMA,
            pltpu.SemaphoreType.DMA,
        ],
    )
    def k(nbr_hbm, fm_hbm, out_hbm, nbr_v, chunk_a, chunk_b, outa,
          sem_a, sem_b):
        wid = lax.axis_index("s") * NC + lax.axis_index("c")
        b = wid // (NW // bs)
        c0 = (wid % (NW // bs)) * CG
        pltpu.sync_copy(nbr_hbm.at[pl.ds(b * (_NEIGHBOR_NUM * P),
                                         _NEIGHBOR_NUM * P)], nbr_v)

        chunks = (chunk_a, chunk_b)
        sems = (sem_a, sem_b)
        rowcs = [jnp.full((16,), c, jnp.int32) for c in range(CG)]

        def src(a):
            return fm_hbm.at[b, a, pl.ds(c0, CG), :]

        cps = [pltpu.async_copy(src(0), chunk_a, sem_a), None]
        for a in range(_ANCHOR):
            cur, nxt = a % 2, (a + 1) % 2
            if a + 1 < _ANCHOR:
                cps[nxt] = pltpu.async_copy(src(a + 1), chunks[nxt],
                                            sems[nxt])
            cps[cur].wait()
            chunk = chunks[cur]

            @plsc.parallel_loop(0, P // 16, unroll=4)
            def qloop(g):
                qb = g * 16
                nb = [nbr_v[pl.ds(n * P + qb, 16)]
                      for n in range(_NEIGHBOR_NUM)]
                for c in range(CG):
                    acc = None
                    for n in range(_NEIGHBOR_NUM):
                        v = plsc.load_gather(chunk, [rowcs[c], nb[n]])
                        acc = v if acc is None else jnp.maximum(acc, v)
                    outa[c, pl.ds(qb, 16)] = acc
            pltpu.sync_copy(outa, out_hbm.at[b, a, pl.ds(c0, CG), :])

    return k(nbr_flat, fm_t)


_SAMPLE_IDX_CACHE = {}


def _sample_idx(V, P):
    if (V, P) not in _SAMPLE_IDX_CACHE:
        try:
            with jax.ensure_compile_time_eval():
                perm = jax.random.permutation(jax.random.key(123), V)[:P]
            _SAMPLE_IDX_CACHE[(V, P)] = np.asarray(perm)
        except Exception:
            return jax.random.permutation(jax.random.key(123), V)[:P]
    return _SAMPLE_IDX_CACHE[(V, P)]


def kernel(vertices, feature_map):
    bs, V, _ = vertices.shape
    C = feature_map.shape[1]
    P = V // _POOLING_RATE
    sample_idx = _sample_idx(V, P)
    vertices_pool = vertices[:, sample_idx, :]
    queries = jnp.transpose(vertices_pool, (0, 2, 1))

    idx5 = _knn_topk_tc(vertices, queries)
    nbr = idx5[:, 1:1 + _NEIGHBOR_NUM, :]
    nbr_flat = nbr.reshape(bs * _NEIGHBOR_NUM * P)

    fm_t = jnp.transpose(feature_map, (0, 3, 1, 2))

    pooled_t = _gather_max_sc(nbr_flat, fm_t, bs, C, V, P)
    feature_map_pool = jnp.transpose(pooled_t, (0, 2, 3, 1))
    return (vertices_pool, feature_map_pool)


# speedup vs baseline: 1.2166x; 1.2166x over previous
import functools

import jax
import jax.numpy as jnp
import numpy as np
from jax import lax
from jax.experimental import pallas as pl
from jax.experimental.pallas import tpu as pltpu
from jax.experimental.pallas import tpu_sc as plsc

_POOLING_RATE = 4
_NEIGHBOR_NUM = 4
_ANCHOR = 12
_ROW = 16


def _knn_topk_tc(vertices, queries):
    bs, V, _ = vertices.shape
    Q = queries.shape[2]
    QB = 512

    def body(v_ref, q_ref, o_ref):
        wx = v_ref[0, :, 0:1]
        wy = v_ref[0, :, 1:2]
        wz = v_ref[0, :, 2:3]
        qx = q_ref[0, 0:1, :]
        qy = q_ref[0, 1:2, :]
        qz = q_ref[0, 2:3, :]
        wn = wx * wx + wy * wy + wz * wz
        qn = qx * qx + qy * qy + qz * qz
        wxb = wx.astype(jnp.bfloat16).astype(jnp.float32)
        wyb = wy.astype(jnp.bfloat16).astype(jnp.float32)
        wzb = wz.astype(jnp.bfloat16).astype(jnp.float32)
        qxb = qx.astype(jnp.bfloat16).astype(jnp.float32)
        qyb = qy.astype(jnp.bfloat16).astype(jnp.float32)
        qzb = qz.astype(jnp.bfloat16).astype(jnp.float32)
        inner = (wxb * qxb + wyb * qyb) + wzb * qzb
        dist = (inner * (-2.0) + wn) + qn
        iota = lax.broadcasted_iota(jnp.int32, (V, QB), 0)
        big = jnp.int32(2 ** 30)
        for k in range(5):
            mval = jnp.min(dist, axis=0, keepdims=True)
            cand = jnp.where(dist == mval, iota, big)
            midx = jnp.min(cand, axis=0, keepdims=True)
            if k > 0:
                o_ref[0, k:k + 1, :] = midx
            if k < 4:
                dist = jnp.where(iota == midx, jnp.float32(jnp.inf), dist)

    return pl.pallas_call(
        body,
        grid=(bs, Q // QB),
        in_specs=[
            pl.BlockSpec((1, V, 3), lambda b, i: (b, 0, 0)),
            pl.BlockSpec((1, 3, QB), lambda b, i: (b, 0, i)),
        ],
        out_specs=pl.BlockSpec((1, 8, QB), lambda b, i: (b, 0, i)),
        out_shape=jax.ShapeDtypeStruct((bs, 8, Q), jnp.int32),
    )(vertices, queries)


def _gather_max_sc(nbr_flat, fm_t, bs, C, V, P):
    info = plsc.get_sparse_core_info()
    NC, NS = info.num_cores, info.num_subcores
    NW = NC * NS
    CG = 8
    mesh = plsc.VectorSubcoreMesh(core_axis_name="c", subcore_axis_name="s")

    @functools.partial(
        pl.kernel,
        mesh=mesh,
        out_type=jax.ShapeDtypeStruct((bs, _ANCHOR, C, P), jnp.float32),
        compiler_params=pltpu.CompilerParams(use_tc_tiling_on_sc=True,
                                             needs_layout_passes=False),
        scratch_types=[
            pltpu.VMEM((_NEIGHBOR_NUM * P,), jnp.int32),
            pltpu.VMEM((CG, V), jnp.float32),
            pltpu.VMEM((CG, V), jnp.float32),
            pltpu.VMEM((CG, P), jnp.float32),
            pltpu.SemaphoreType.DMA,
            pltpu.SemaphoreType.DMA,
        ],
    )
    def k(nbr_hbm, fm_hbm, out_hbm, nbr_v, chunk_a, chunk_b, outa,
          sem_a, sem_b):
        wid = lax.axis_index("s") * NC + lax.axis_index("c")
        b = wid // (NW // bs)
        c0 = (wid % (NW // bs)) * CG
        pltpu.sync_copy(nbr_hbm.at[pl.ds(b * (_NEIGHBOR_NUM * P),
                                         _NEIGHBOR_NUM * P)], nbr_v)

        chunks = (chunk_a, chunk_b)
        sems = (sem_a, sem_b)
        rowcs = [jnp.full((16,), c, jnp.int32) for c in range(CG)]

        def src(a):
            return fm_hbm.at[b, a, pl.ds(c0, CG), :]

        cps = [pltpu.async_copy(src(0), chunk_a, sem_a), None]
        for a in range(_ANCHOR):
            cur, nxt = a % 2, (a + 1) % 2
            if a + 1 < _ANCHOR:
                cps[nxt] = pltpu.async_copy(src(a + 1), chunks[nxt],
                                            sems[nxt])
            cps[cur].wait()
            chunk = chunks[cur]

            @plsc.parallel_loop(0, P // 16, unroll=2)
            def qloop(g):
                qb = g * 16
                nb = [nbr_v[pl.ds(n * P + qb, 16)]
                      for n in range(_NEIGHBOR_NUM)]
                for c in range(CG):
                    acc = None
                    for n in range(_NEIGHBOR_NUM):
                        v = plsc.load_gather(chunk, [rowcs[c], nb[n]])
                        acc = v if acc is None else jnp.maximum(acc, v)
                    outa[c, pl.ds(qb, 16)] = acc
            pltpu.sync_copy(outa, out_hbm.at[b, a, pl.ds(c0, CG), :])

    return k(nbr_flat, fm_t)


_SAMPLE_IDX_CACHE = {}


def _sample_idx(V, P):
    if (V, P) not in _SAMPLE_IDX_CACHE:
        try:
            with jax.ensure_compile_time_eval():
                perm = jax.random.permutation(jax.random.key(123), V)[:P]
            _SAMPLE_IDX_CACHE[(V, P)] = np.asarray(perm)
        except Exception:
            return jax.random.permutation(jax.random.key(123), V)[:P]
    return _SAMPLE_IDX_CACHE[(V, P)]


def kernel(vertices, feature_map):
    bs, V, _ = vertices.shape
    C = feature_map.shape[1]
    P = V // _POOLING_RATE
    sample_idx = _sample_idx(V, P)
    vertices_pool = vertices[:, sample_idx, :]
    queries = jnp.transpose(vertices_pool, (0, 2, 1))

    idx5 = _knn_topk_tc(vertices, queries)
    nbr = idx5[:, 1:1 + _NEIGHBOR_NUM, :]
    nbr_flat = nbr.reshape(bs * _NEIGHBOR_NUM * P)

    fm_t = jnp.transpose(feature_map, (0, 3, 1, 2))

    pooled_t = _gather_max_sc(nbr_flat, fm_t, bs, C, V, P)
    feature_map_pool = jnp.transpose(pooled_t, (0, 2, 3, 1))
    return (vertices_pool, feature_map_pool)


# speedup vs baseline: 1.2541x; 1.0308x over previous
import functools

import jax
import jax.numpy as jnp
import numpy as np
from jax import lax
from jax.experimental import pallas as pl
from jax.experimental.pallas import tpu as pltpu
from jax.experimental.pallas import tpu_sc as plsc

_POOLING_RATE = 4
_NEIGHBOR_NUM = 4
_ANCHOR = 12
_ROW = 16


def _knn_topk_tc(vertices, queries):
    bs, V, _ = vertices.shape
    Q = queries.shape[2]
    QB = 1024

    def body(v_ref, q_ref, o_ref):
        wx = v_ref[0, :, 0:1]
        wy = v_ref[0, :, 1:2]
        wz = v_ref[0, :, 2:3]
        qx = q_ref[0, 0:1, :]
        qy = q_ref[0, 1:2, :]
        qz = q_ref[0, 2:3, :]
        wn = wx * wx + wy * wy + wz * wz
        qn = qx * qx + qy * qy + qz * qz
        wxb = wx.astype(jnp.bfloat16).astype(jnp.float32)
        wyb = wy.astype(jnp.bfloat16).astype(jnp.float32)
        wzb = wz.astype(jnp.bfloat16).astype(jnp.float32)
        qxb = qx.astype(jnp.bfloat16).astype(jnp.float32)
        qyb = qy.astype(jnp.bfloat16).astype(jnp.float32)
        qzb = qz.astype(jnp.bfloat16).astype(jnp.float32)
        inner = (wxb * qxb + wyb * qyb) + wzb * qzb
        dist = (inner * (-2.0) + wn) + qn
        iota = lax.broadcasted_iota(jnp.int32, (V, QB), 0)
        big = jnp.int32(2 ** 30)
        for k in range(5):
            mval = jnp.min(dist, axis=0, keepdims=True)
            cand = jnp.where(dist == mval, iota, big)
            midx = jnp.min(cand, axis=0, keepdims=True)
            if k > 0:
                o_ref[0, k:k + 1, :] = midx
            if k < 4:
                dist = jnp.where(iota == midx, jnp.float32(jnp.inf), dist)

    return pl.pallas_call(
        body,
        grid=(bs, Q // QB),
        in_specs=[
            pl.BlockSpec((1, V, 3), lambda b, i: (b, 0, 0)),
            pl.BlockSpec((1, 3, QB), lambda b, i: (b, 0, i)),
        ],
        out_specs=pl.BlockSpec((1, 8, QB), lambda b, i: (b, 0, i)),
        out_shape=jax.ShapeDtypeStruct((bs, 8, Q), jnp.int32),
    )(vertices, queries)


def _gather_max_sc(nbr_flat, fm_t, bs, C, V, P):
    info = plsc.get_sparse_core_info()
    NC, NS = info.num_cores, info.num_subcores
    NW = NC * NS
    CG = 8
    mesh = plsc.VectorSubcoreMesh(core_axis_name="c", subcore_axis_name="s")

    @functools.partial(
        pl.kernel,
        mesh=mesh,
        out_type=jax.ShapeDtypeStruct((bs, _ANCHOR, C, P), jnp.float32),
        compiler_params=pltpu.CompilerParams(use_tc_tiling_on_sc=True,
                                             needs_layout_passes=False),
        scratch_types=[
            pltpu.VMEM((_NEIGHBOR_NUM * P,), jnp.int32),
            pltpu.VMEM((CG, V), jnp.float32),
            pltpu.VMEM((CG, V), jnp.float32),
            pltpu.VMEM((CG, P), jnp.float32),
            pltpu.SemaphoreType.DMA,
            pltpu.SemaphoreType.DMA,
        ],
    )
    def k(nbr_hbm, fm_hbm, out_hbm, nbr_v, chunk_a, chunk_b, outa,
          sem_a, sem_b):
        wid = lax.axis_index("s") * NC + lax.axis_index("c")
        b = wid // (NW // bs)
        c0 = (wid % (NW // bs)) * CG
        pltpu.sync_copy(nbr_hbm.at[pl.ds(b * (_NEIGHBOR_NUM * P),
                                         _NEIGHBOR_NUM * P)], nbr_v)

        chunks = (chunk_a, chunk_b)
        sems = (sem_a, sem_b)
        rowcs = [jnp.full((16,), c, jnp.int32) for c in range(CG)]

        def src(a):
            return fm_hbm.at[b, a, pl.ds(c0, CG), :]

        cps = [pltpu.async_copy(src(0), chunk_a, sem_a), None]
        for a in range(_ANCHOR):
            cur, nxt = a % 2, (a + 1) % 2
            if a + 1 < _ANCHOR:
                cps[nxt] = pltpu.async_copy(src(a + 1), chunks[nxt],
                                            sems[nxt])
            cps[cur].wait()
            chunk = chunks[cur]

            @plsc.parallel_loop(0, P // 16, unroll=2)
            def qloop(g):
                qb = g * 16
                nb = [nbr_v[pl.ds(n * P + qb, 16)]
                      for n in range(_NEIGHBOR_NUM)]
                for c in range(CG):
                    acc = None
                    for n in range(_NEIGHBOR_NUM):
                        v = plsc.load_gather(chunk, [rowcs[c], nb[n]])
                        acc = v if acc is None else jnp.maximum(acc, v)
                    outa[c, pl.ds(qb, 16)] = acc
            pltpu.sync_copy(outa, out_hbm.at[b, a, pl.ds(c0, CG), :])

    return k(nbr_flat, fm_t)


_SAMPLE_IDX_CACHE = {}


def _sample_idx(V, P):
    if (V, P) not in _SAMPLE_IDX_CACHE:
        try:
            with jax.ensure_compile_time_eval():
                perm = jax.random.permutation(jax.random.key(123), V)[:P]
            _SAMPLE_IDX_CACHE[(V, P)] = np.asarray(perm)
        except Exception:
            return jax.random.permutation(jax.random.key(123), V)[:P]
    return _SAMPLE_IDX_CACHE[(V, P)]


def kernel(vertices, feature_map):
    bs, V, _ = vertices.shape
    C = feature_map.shape[1]
    P = V // _POOLING_RATE
    sample_idx = _sample_idx(V, P)
    vertices_pool = vertices[:, sample_idx, :]
    queries = jnp.transpose(vertices_pool, (0, 2, 1))

    idx5 = _knn_topk_tc(vertices, queries)
    nbr = idx5[:, 1:1 + _NEIGHBOR_NUM, :]
    nbr_flat = nbr.reshape(bs * _NEIGHBOR_NUM * P)

    fm_t = jnp.transpose(feature_map, (0, 3, 1, 2))

    pooled_t = _gather_max_sc(nbr_flat, fm_t, bs, C, V, P)
    feature_map_pool = jnp.transpose(pooled_t, (0, 2, 3, 1))
    return (vertices_pool, feature_map_pool)
